# baseline (speedup 1.0000x reference)
import math

import jax
import jax.numpy as jnp
from jax.experimental import pallas as pl
from jax.experimental.pallas import tpu as pltpu

_TILE = 8192
_NEG = -1e30


def _attn_kernel(starts_ref, ends_ref, enc_ref, wqt_ref, bq_ref, wkt_ref,
                 bk_ref, wvt_ref, bv_ref, soc_a_ref, soc_b_ref, out_ref,
                 q_ref, m1_ref, s1_ref, acc1_ref, m2_ref, s2_ref, acc2_ref):
    j = pl.program_id(0)
    nt = pl.num_programs(0)
    b, d = out_ref.shape
    tile = soc_a_ref.shape[1]
    half = nt * tile

    @pl.when(j == 0)
    def _init():
        q = jnp.dot(enc_ref[...], wqt_ref[...],
                    preferred_element_type=jnp.float32) + bq_ref[...]
        q_ref[...] = jnp.maximum(q, 0.0) * (1.0 / math.sqrt(d))
        for mr, sr, ar in ((m1_ref, s1_ref, acc1_ref),
                           (m2_ref, s2_ref, acc2_ref)):
            mr[...] = jnp.full((b, d), _NEG, jnp.float32)
            sr[...] = jnp.zeros((b, d), jnp.float32)
            ar[...] = jnp.zeros((b, d), jnp.float32)

    max_end = jnp.max(ends_ref[...])

    def _process(soc_ref, base, m_ref, s_ref, acc_ref):
        @pl.when(base < max_end)
        def _tile():
            tok = soc_ref[0]
            k = jnp.maximum(
                jnp.dot(tok, wkt_ref[...],
                        preferred_element_type=jnp.float32) + bk_ref[...], 0.0)
            v = jnp.maximum(
                jnp.dot(tok, wvt_ref[...],
                        preferred_element_type=jnp.float32) + bv_ref[...], 0.0)

            logits = jax.lax.dot_general(
                q_ref[...], k, (((1,), (1,)), ((), ())),
                preferred_element_type=jnp.float32)
            col = base + jax.lax.broadcasted_iota(jnp.int32, (b, tile), 1)
            mask = (col >= starts_ref[...]) & (col < ends_ref[...])
            logits = jnp.where(mask, logits, _NEG)

            m_old = m_ref[...]
            row_max = jnp.max(logits, axis=1, keepdims=True)
            m_new = jnp.maximum(m_old, row_max)
            alpha = jnp.exp(m_old - m_new)
            p = jnp.exp(logits - jnp.max(m_new, axis=1, keepdims=True))

            s_ref[...] = s_ref[...] * alpha + jnp.sum(p, axis=1, keepdims=True)
            acc_ref[...] = acc_ref[...] * alpha + jnp.dot(
                p, v, preferred_element_type=jnp.float32)
            m_ref[...] = m_new

    _process(soc_a_ref, j * tile, m1_ref, s1_ref, acc1_ref)
    _process(soc_b_ref, half + j * tile, m2_ref, s2_ref, acc2_ref)

    @pl.when(j == nt - 1)
    def _fin():
        m1, m2 = m1_ref[...], m2_ref[...]
        m = jnp.maximum(m1, m2)
        w1 = jnp.exp(m1 - m)
        w2 = jnp.exp(m2 - m)
        s = s1_ref[...] * w1 + s2_ref[...] * w2
        out_ref[...] = (acc1_ref[...] * w1 + acc2_ref[...] * w2) / s


def kernel(enc_hidden, social_ht, neighbors_idx_start, neighbors_idx_end,
           Wq, bq, Wk, bk, Wv, bv):
    b, d = enc_hidden.shape
    t = social_ht.shape[0]
    half = t // 2
    nt = half // _TILE

    starts = neighbors_idx_start.astype(jnp.int32).reshape(b, 1)
    ends = neighbors_idx_end.astype(jnp.int32).reshape(b, 1)
    social3 = social_ht.reshape(2, half, d)

    const = lambda j: (0, 0)
    out = pl.pallas_call(
        _attn_kernel,
        grid=(nt,),
        in_specs=[
            pl.BlockSpec((b, 1), const),
            pl.BlockSpec((b, 1), const),
            pl.BlockSpec((b, d), const),
            pl.BlockSpec((d, d), const),
            pl.BlockSpec((1, d), const),
            pl.BlockSpec((d, d), const),
            pl.BlockSpec((1, d), const),
            pl.BlockSpec((d, d), const),
            pl.BlockSpec((1, d), const),
            pl.BlockSpec((1, _TILE, d), lambda j: (0, j, 0)),
            pl.BlockSpec((1, _TILE, d), lambda j: (1, j, 0)),
        ],
        out_specs=pl.BlockSpec((b, d), const),
        out_shape=jax.ShapeDtypeStruct((b, d), jnp.float32),
        scratch_shapes=[pltpu.VMEM((b, d), jnp.float32)] * 7,
        compiler_params=pltpu.CompilerParams(
            dimension_semantics=("arbitrary",)),
    )(starts, ends, enc_hidden,
      Wq.T, bq.reshape(1, d),
      Wk.T, bk.reshape(1, d),
      Wv.T, bv.reshape(1, d), social3, social3)
    return out
